# Initial kernel scaffold; baseline (speedup 1.0000x reference)
import jax
import jax.numpy as jnp
from jax.experimental import pallas as pl


def kernel(x, edge_index, batch, W1, b1, Wp1, bp1, W2, b2, Wp2, bp2, W3, b3, Wf1, bf1, Wf2, bf2):
    raise NotImplementedError("write your pallas kernel here")


# speedup vs baseline: 2.2767x; 2.2767x over previous
import functools
import math

import jax
import jax.numpy as jnp
from jax import lax
from jax.experimental import pallas as pl
from jax.experimental.pallas import tpu as pltpu
from jax.experimental.pallas import tpu_sc as plsc

NC = 2
NS = 16
NW = NC * NS
CH = 128
SLOPE = 0.01


def _rup(a, b):
  return (a + b - 1) // b * b


def _mesh():
  return plsc.VectorSubcoreMesh(
      core_axis_name="c", subcore_axis_name="s", num_cores=NC, num_subcores=NS)


@functools.lru_cache(maxsize=None)
def _make_sagg(n_acc, e_pad, gather):
  per_w = e_pad // NW
  n_ch = per_w // CH
  tile_rows = n_acc // NS

  def body(*refs):
    if gather:
      tab_hbm, src_hbm, dst_hbm, out_hbm, acc, zb, rows, sidx, didx, sem = refs
    else:
      dst_hbm, out_hbm, acc, zb, rows, sidx, didx, sem = refs
    c = lax.axis_index("c")
    s = lax.axis_index("s")
    wid = s * NC + c

    def zloop(i, carry):
      zb[pl.ds(i * 16, 16)] = jnp.zeros((16,), jnp.float32)
      return carry

    lax.fori_loop(0, zb.shape[0] // 16, zloop, 0)
    if not gather:
      for j in range(CH // 16):
        rows[pl.ds(j * 16, 16)] = jnp.ones((16,), jnp.float32)
    r0 = s * tile_rows
    pltpu.sync_copy(zb.at[pl.ds(0, tile_rows)], acc.at[pl.ds(r0, tile_rows)])
    plsc.subcore_barrier()

    def eloop(i, carry):
      base = wid * per_w + i * CH
      pltpu.sync_copy(dst_hbm.at[pl.ds(base, CH)], didx.at[0])
      if gather:
        pltpu.sync_copy(src_hbm.at[pl.ds(base, CH)], sidx)
        pltpu.async_copy(tab_hbm.at[sidx], rows, sem).wait()
      pltpu.sync_copy(rows, acc.at[didx.at[0]], add=True)
      return carry

    lax.fori_loop(0, n_ch, eloop, 0)
    plsc.subcore_barrier()
    pltpu.sync_copy(acc.at[pl.ds(r0, tile_rows)], zb.at[pl.ds(0, tile_rows)])
    pltpu.sync_copy(zb.at[pl.ds(0, tile_rows)],
                    out_hbm.at[pl.ds(c * n_acc + r0, tile_rows)])

  return pl.kernel(
      body,
      out_type=jax.ShapeDtypeStruct((NC * n_acc,), jnp.float32),
      mesh=_mesh(),
      compiler_params=pltpu.CompilerParams(needs_layout_passes=False),
      scratch_types=[
          pltpu.VMEM_SHARED((n_acc,), jnp.float32),
          pltpu.VMEM((_rup(tile_rows, 16),), jnp.float32),
          pltpu.VMEM((CH,), jnp.float32),
          pltpu.VMEM((CH,), jnp.int32),
          pltpu.VMEM((1, CH), jnp.int32),
          pltpu.SemaphoreType.DMA,
      ],
  )


@functools.lru_cache(maxsize=None)
def _make_agg(n_acc, e_pad):
  per_w = e_pad // NW
  n_ch = per_w // CH
  tile_rows = n_acc // NS

  def body(tab_hbm, src_hbm, dst_hbm, out_hbm, acc, zb, rows, sidx, didx, sem):
    c = lax.axis_index("c")
    s = lax.axis_index("s")
    wid = s * NC + c

    def zloop(i, carry):
      for j in range(8):
        zb[i, pl.ds(j * 16, 16)] = jnp.zeros((16,), jnp.float32)
      return carry

    lax.fori_loop(0, zb.shape[0], zloop, 0)
    r0 = s * tile_rows
    nfull = tile_rows // 128
    rem = tile_rows - nfull * 128

    def zcopy(i, carry):
      pltpu.sync_copy(zb, acc.at[pl.ds(r0 + i * 128, 128)])
      return carry

    lax.fori_loop(0, nfull, zcopy, 0)
    if rem:
      pltpu.sync_copy(zb.at[pl.ds(0, rem)],
                      acc.at[pl.ds(r0 + nfull * 128, rem)])
    plsc.subcore_barrier()

    def eloop(i, carry):
      base = wid * per_w + i * CH
      pltpu.sync_copy(src_hbm.at[pl.ds(base, CH)], sidx)
      pltpu.sync_copy(dst_hbm.at[pl.ds(base, CH)], didx.at[0])
      pltpu.async_copy(tab_hbm.at[sidx], rows, sem).wait()
      pltpu.sync_copy(rows, acc.at[didx.at[0]], add=True)
      return carry

    lax.fori_loop(0, n_ch, eloop, 0)
    plsc.subcore_barrier()

    def ocopy(i, carry):
      pltpu.sync_copy(acc.at[pl.ds(r0 + i * 128, 128)], zb)
      pltpu.sync_copy(zb, out_hbm.at[c, pl.ds(r0 + i * 128, 128)])
      return carry

    lax.fori_loop(0, nfull, ocopy, 0)
    if rem:
      pltpu.sync_copy(acc.at[pl.ds(r0 + nfull * 128, rem)],
                      zb.at[pl.ds(0, rem)])
      pltpu.sync_copy(zb.at[pl.ds(0, rem)],
                      out_hbm.at[c, pl.ds(r0 + nfull * 128, rem)])

  return pl.kernel(
      body,
      out_type=jax.ShapeDtypeStruct((NC, n_acc, 128), jnp.float32),
      mesh=_mesh(),
      compiler_params=pltpu.CompilerParams(needs_layout_passes=False),
      scratch_types=[
          pltpu.VMEM_SHARED((n_acc, 128), jnp.float32),
          pltpu.VMEM((128, 128), jnp.float32),
          pltpu.VMEM((CH, 128), jnp.float32),
          pltpu.VMEM((CH,), jnp.int32),
          pltpu.VMEM((1, CH), jnp.int32),
          pltpu.SemaphoreType.DMA,
      ],
  )


@functools.lru_cache(maxsize=None)
def _make_pool(n_acc, k_pad, e_pad, trash):
  per_w = e_pad // NW
  n_ch = per_w // CH
  RCH = 80
  per_w_rows = k_pad // NW
  r_ch = per_w_rows // RCH
  assert per_w_rows % RCH == 0

  def body(ht_hbm, perm_hbm, keep_hbm, nid_hbm, src_hbm, dst_hbm,
           x_out, srcn_hbm, dstn_hbm,
           keep_v, nid_v, rowbuf, pidx, sidx, didx, sob, dob, sem):
    c = lax.axis_index("c")
    s = lax.axis_index("s")
    wid = s * NC + c
    pltpu.sync_copy(keep_hbm, keep_v)
    pltpu.sync_copy(nid_hbm, nid_v)

    def rloop(j, carry):
      base = wid * per_w_rows + j * RCH
      pltpu.sync_copy(perm_hbm.at[pl.ds(base, RCH)], pidx)
      pltpu.async_copy(ht_hbm.at[pidx], rowbuf, sem).wait()
      pltpu.sync_copy(rowbuf, x_out.at[pl.ds(base, RCH)])
      return carry

    lax.fori_loop(0, r_ch, rloop, 0)

    zero16 = jnp.zeros((16,), jnp.int32)
    trash16 = jnp.full((16,), trash, jnp.int32)

    def eloop(i, carry):
      base = wid * per_w + i * CH
      pltpu.sync_copy(src_hbm.at[pl.ds(base, CH)], sidx)
      pltpu.sync_copy(dst_hbm.at[pl.ds(base, CH)], didx)
      for g in range(CH // 16):
        sv = sidx[pl.ds(g * 16, 16)]
        dv = didx[pl.ds(g * 16, 16)]
        ks = plsc.load_gather(keep_v, [sv])
        kd = plsc.load_gather(keep_v, [dv])
        ns_ = plsc.load_gather(nid_v, [sv])
        nd_ = plsc.load_gather(nid_v, [dv])
        valid = (ks + kd) == 2
        sob[pl.ds(g * 16, 16)] = jnp.where(valid, ns_, zero16)
        dob[pl.ds(g * 16, 16)] = jnp.where(valid, nd_, trash16)
      pltpu.sync_copy(sob, srcn_hbm.at[pl.ds(base, CH)])
      pltpu.sync_copy(dob, dstn_hbm.at[pl.ds(base, CH)])
      return carry

    lax.fori_loop(0, n_ch, eloop, 0)

  return pl.kernel(
      body,
      out_type=[
          jax.ShapeDtypeStruct((k_pad, 128), jnp.float32),
          jax.ShapeDtypeStruct((e_pad,), jnp.int32),
          jax.ShapeDtypeStruct((e_pad,), jnp.int32),
      ],
      mesh=_mesh(),
      compiler_params=pltpu.CompilerParams(needs_layout_passes=False),
      scratch_types=[
          pltpu.VMEM((n_acc,), jnp.int32),
          pltpu.VMEM((n_acc,), jnp.int32),
          pltpu.VMEM((RCH, 128), jnp.float32),
          pltpu.VMEM((RCH,), jnp.int32),
          pltpu.VMEM((CH,), jnp.int32),
          pltpu.VMEM((CH,), jnp.int32),
          pltpu.VMEM((CH,), jnp.int32),
          pltpu.VMEM((CH,), jnp.int32),
          pltpu.SemaphoreType.DMA,
      ],
  )


def _tc_pre(degp, h, w):
  rows = h.shape[0]

  def body(deg_ref, h_ref, w_ref, hw_ref, g_ref, dis_ref):
    deg = deg_ref[0, :rows] + deg_ref[1, :rows] + 1.0
    dis = lax.rsqrt(deg)
    hw = jnp.dot(h_ref[...], w_ref[...], preferred_element_type=jnp.float32)
    hw_ref[...] = hw
    g_ref[...] = dis * hw
    dis_ref[...] = dis

  return pl.pallas_call(
      body,
      out_shape=[
          jax.ShapeDtypeStruct((rows, 128), jnp.float32),
          jax.ShapeDtypeStruct((rows, 128), jnp.float32),
          jax.ShapeDtypeStruct((rows, 1), jnp.float32),
      ])(degp, h, w)


def _tc_post(aggp, dis, hw, b, wp):
  rows = hw.shape[0]

  def body(agg_ref, dis_ref, hw_ref, b_ref, wp_ref, h_ref, hs_ref, gs_ref):
    dis = dis_ref[...]
    acc = agg_ref[0, :rows] + agg_ref[1, :rows]
    z = dis * acc + dis * dis * hw_ref[...] + b_ref[...]
    h = jnp.where(z >= 0, z, SLOPE * z)
    h_ref[...] = h
    hs = jnp.dot(h, wp_ref[...], preferred_element_type=jnp.float32)
    hs_ref[...] = hs
    gs_ref[...] = dis * hs

  return pl.pallas_call(
      body,
      out_shape=[
          jax.ShapeDtypeStruct((rows, 128), jnp.float32),
          jax.ShapeDtypeStruct((rows, 1), jnp.float32),
          jax.ShapeDtypeStruct((rows, 1), jnp.float32),
      ])(aggp, dis, hw, b.reshape(1, 128), wp)


def _tc_score(saggp, dis, hs, bp, h):
  rows = h.shape[0]

  def body(sagg_ref, dis_ref, hs_ref, bp_ref, h_ref, ht_ref, sc_ref):
    dis = dis_ref[...]
    sacc = sagg_ref[0, :rows] + sagg_ref[1, :rows]
    score = dis * sacc + dis * dis * hs_ref[...] + bp_ref[...]
    t = jnp.tanh(score)
    ht_ref[...] = h_ref[...] * t
    sc_ref[...] = score

  return pl.pallas_call(
      body,
      out_shape=[
          jax.ShapeDtypeStruct((rows, 128), jnp.float32),
          jax.ShapeDtypeStruct((rows, 1), jnp.float32),
      ])(saggp, dis, hs, bp.reshape(1, 1), h)


def _tc_head(aggp, dis, hw, b, wf1, bf1, wf2, bf2, n_real):
  rows = hw.shape[0]
  c_out = wf2.shape[1]

  def body(agg_ref, dis_ref, hw_ref, b_ref, wf1_ref, bf1_ref, wf2_ref,
           bf2_ref, out_ref):
    dis = dis_ref[...]
    acc = agg_ref[0, :rows] + agg_ref[1, :rows]
    z = dis * acc + dis * dis * hw_ref[...] + b_ref[...]
    h = jnp.where(z >= 0, z, SLOPE * z)
    rid = lax.broadcasted_iota(jnp.int32, (rows, 128), 0)
    msk = rid < n_real
    hsum = jnp.sum(jnp.where(msk, h, 0.0), axis=0, keepdims=True)
    hmax = jnp.max(jnp.where(msk, h, -1e30), axis=0, keepdims=True)
    gcat = jnp.concatenate([hsum / n_real, hmax], axis=1)
    z1 = jnp.dot(gcat, wf1_ref[...], preferred_element_type=jnp.float32)
    z1 = z1 + bf1_ref[...]
    z1 = jnp.where(z1 >= 0, z1, SLOPE * z1)
    out_ref[...] = jnp.dot(z1, wf2_ref[...],
                           preferred_element_type=jnp.float32) + bf2_ref[...]

  return pl.pallas_call(
      body,
      out_shape=jax.ShapeDtypeStruct((1, c_out), jnp.float32),
  )(aggp, dis, hw, b.reshape(1, 128), wf1, bf1.reshape(1, 128), wf2,
    bf2.reshape(1, c_out))


def kernel(x, edge_index, batch, W1, b1, Wp1, bp1, W2, b2, Wp2, bp2, W3, b3,
           Wf1, bf1, Wf2, bf2):
  n1 = x.shape[0]
  e = edge_index.shape[1]
  e_pad = _rup(e, NW * CH)
  k1 = math.ceil(0.5 * n1)
  k2 = math.ceil(0.5 * k1)
  na1 = _rup(n1 + 1, 128)
  na2 = _rup(k1 + 1, 128)
  na3 = _rup(k2 + 1, 128)
  pad = e_pad - e
  src = jnp.concatenate([edge_index[0], jnp.zeros((pad,), jnp.int32)])
  dst = jnp.concatenate([edge_index[1], jnp.full((pad,), n1, jnp.int32)])

  degp = _make_sagg(na1, e_pad, False)(dst)
  hw1, g1, dis1 = _tc_pre(degp.reshape(NC, na1, 1), x, W1)
  aggp = _make_agg(na1, e_pad)(g1, src, dst)
  h1, hs1, gs1 = _tc_post(aggp, dis1, hw1, b1, Wp1)
  saggp = _make_sagg(na1, e_pad, True)(gs1.reshape(n1), src, dst)
  ht1, sc1 = _tc_score(saggp.reshape(NC, na1, 1), dis1, hs1, bp1, h1)
  _, perm1 = lax.top_k(sc1.reshape(n1), k1)
  perm1 = perm1.astype(jnp.int32)
  keep1 = jnp.zeros((na1,), jnp.int32).at[perm1].set(1)
  nid1 = jnp.zeros((na1,), jnp.int32).at[perm1].set(
      jnp.arange(k1, dtype=jnp.int32))
  perm1p = jnp.concatenate([perm1, jnp.zeros((na2 - k1,), jnp.int32)])
  x2, src2, dst2 = _make_pool(na1, na2, e_pad, k1)(
      ht1, perm1p, keep1, nid1, src, dst)

  degp2 = _make_sagg(na2, e_pad, False)(dst2)
  hw2, g2, dis2 = _tc_pre(degp2.reshape(NC, na2, 1), x2, W2)
  aggp2 = _make_agg(na2, e_pad)(g2, src2, dst2)
  h2, hs2, gs2 = _tc_post(aggp2, dis2, hw2, b2, Wp2)
  saggp2 = _make_sagg(na2, e_pad, True)(gs2.reshape(na2), src2, dst2)
  ht2, sc2 = _tc_score(saggp2.reshape(NC, na2, 1), dis2, hs2, bp2, h2)
  _, perm2 = lax.top_k(sc2.reshape(na2)[:k1], k2)
  perm2 = perm2.astype(jnp.int32)
  keep2 = jnp.zeros((na2,), jnp.int32).at[perm2].set(1)
  nid2 = jnp.zeros((na2,), jnp.int32).at[perm2].set(
      jnp.arange(k2, dtype=jnp.int32))
  perm2p = jnp.concatenate([perm2, jnp.zeros((na3 - k2,), jnp.int32)])
  x3, src3, dst3 = _make_pool(na2, na3, e_pad, k2)(
      ht2, perm2p, keep2, nid2, src2, dst2)

  degp3 = _make_sagg(na3, e_pad, False)(dst3)
  hw3, g3, dis3 = _tc_pre(degp3.reshape(NC, na3, 1), x3, W3)
  aggp3 = _make_agg(na3, e_pad)(g3, src3, dst3)
  return _tc_head(aggp3, dis3, hw3, b3, Wf1, bf1, Wf2, bf2, k2)
